# baseline (speedup 1.0000x reference)
import functools

import jax
import jax.numpy as jnp
from jax import lax
from jax.experimental import pallas as pl
from jax.experimental.pallas import tpu as pltpu
from jax.experimental.pallas import tpu_sc as plsc

B = 16384
L = 50
VOCAB = 1000
EMB = 64

NUM_CORES = 2
NUM_SUBCORES = 16
NW = NUM_CORES * NUM_SUBCORES
BPW = B // NW
CH = 64
NCHUNK = BPW // CH


def _emb_bag(text, table_packed):
    mesh = plsc.VectorSubcoreMesh(core_axis_name="c", subcore_axis_name="s")

    @functools.partial(
        pl.kernel,
        mesh=mesh,
        out_type=jax.ShapeDtypeStruct((B, EMB), jnp.float32),
        scratch_types=[
            pltpu.VMEM((VOCAB * EMB // 2,), jnp.int32),
            pltpu.VMEM((2, CH, L), jnp.int32),
            pltpu.VMEM((2, CH, EMB), jnp.float32),
            pltpu.SemaphoreType.DMA,
            pltpu.SemaphoreType.DMA,
            pltpu.SemaphoreType.DMA,
            pltpu.SemaphoreType.DMA,
            pltpu.SemaphoreType.DMA,
        ],
    )
    def k(text_hbm, table_hbm, out_hbm, tab_v, txt_v, out_v,
          sem_tab, sin0, sin1, sout0, sout1):
        wid = lax.axis_index("s") * NUM_CORES + lax.axis_index("c")
        base = wid * BPW
        sins = (sin0, sin1)
        souts = (sout0, sout1)

        def in_copy(ci, buf):
            return pltpu.make_async_copy(
                text_hbm.at[pl.ds(base + ci * CH, CH)], txt_v.at[buf], sins[buf])

        def out_copy(ci, buf):
            return pltpu.make_async_copy(
                out_v.at[buf], out_hbm.at[pl.ds(base + ci * CH, CH)], souts[buf])

        tab_copy = pltpu.make_async_copy(table_hbm, tab_v, sem_tab)
        tab_copy.start()
        in_copy(0, 0).start()
        in_copy(1, 1).start()
        tab_copy.wait()

        def bag_body(buf):
            def body(b):
                toks = [txt_v[buf, b, pl.ds(c, 16)] for c in (0, 16, 32, 34)]
                acc = [jnp.zeros((16,), jnp.float32) for _ in range(8)]
                for j in range(L):
                    t = toks[j // 16][j % 16] if j < 48 else toks[3][j - 34]
                    off = t * (EMB // 2)
                    lo = tab_v[pl.ds(off, 16)]
                    hi = tab_v[pl.ds(off + 16, 16)]
                    p = (j & 1) * 4
                    acc[p] = acc[p] + lax.bitcast_convert_type(lo << 16, jnp.float32)
                    acc[p + 2] = acc[p + 2] + lax.bitcast_convert_type(lo, jnp.float32)
                    acc[p + 1] = acc[p + 1] + lax.bitcast_convert_type(hi << 16, jnp.float32)
                    acc[p + 3] = acc[p + 3] + lax.bitcast_convert_type(hi, jnp.float32)
                s = jnp.float32(1.0 / L)
                out_v[buf, b, pl.ds(0, 16)] = (acc[0] + acc[4]) * s
                out_v[buf, b, pl.ds(16, 16)] = (acc[1] + acc[5]) * s
                out_v[buf, b, pl.ds(32, 16)] = (acc[2] + acc[6]) * s
                out_v[buf, b, pl.ds(48, 16)] = (acc[3] + acc[7]) * s
            return body

        def pair_body(i, _):
            for buf in range(2):
                ci = 2 * i + buf
                in_copy(ci, buf).wait()

                @pl.when(i > 0)
                def _():
                    out_copy(2 * (i - 1) + buf, buf).wait()

                plsc.parallel_loop(0, CH)(bag_body(buf))
                out_copy(ci, buf).start()

                @pl.when(i < NCHUNK // 2 - 1)
                def _():
                    in_copy(ci + 2, buf).start()
            return 0

        lax.fori_loop(0, NCHUNK // 2, pair_body, 0)
        out_copy(NCHUNK - 2, 0).wait()
        out_copy(NCHUNK - 1, 1).wait()

    return k(text, table_packed)


BLK = 4096


def _mlp_body(emb_ref, num_ref, w1a_ref, w1b_ref, b1_ref, w2_ref, b2_ref,
              w3_ref, b3_ref, out_ref):
    bf = jnp.bfloat16
    h = jnp.dot(emb_ref[...].astype(bf), w1a_ref[...].astype(bf),
                preferred_element_type=jnp.float32)
    h = h + jnp.dot(num_ref[...], w1b_ref[...], preferred_element_type=jnp.float32)
    h = jnp.maximum(h + b1_ref[...], 0.0)
    h = jnp.maximum(
        jnp.dot(h.astype(bf), w2_ref[...].astype(bf),
                preferred_element_type=jnp.float32) + b2_ref[...], 0.0)
    out_ref[...] = (
        jnp.dot(h.astype(bf), w3_ref[...].astype(bf),
                preferred_element_type=jnp.float32) + b3_ref[...])


def _mlp(embedded, numeric, W1a, W1b, b1, W2, b2, W3, b3):
    grid = (B // BLK,)
    full = lambda shape: pl.BlockSpec(shape, lambda i: (0, 0))
    return pl.pallas_call(
        _mlp_body,
        grid=grid,
        in_specs=[
            pl.BlockSpec((BLK, EMB), lambda i: (i, 0)),
            pl.BlockSpec((BLK, 2), lambda i: (i, 0)),
            full((EMB, 128)),
            full((2, 128)),
            full((1, 128)),
            full((128, 64)),
            full((1, 64)),
            full((64, 32)),
            full((1, 32)),
        ],
        out_specs=pl.BlockSpec((BLK, 32), lambda i: (i, 0)),
        out_shape=jax.ShapeDtypeStruct((B, 32), jnp.float32),
    )(embedded, numeric, W1a, W1b, b1, W2, b2, W3, b3)


def kernel(text, numeric_features, table, W1, b1, W2, b2, W3, b3):
    text = text.astype(jnp.int32)
    lo = lax.bitcast_convert_type(
        table[:, :EMB // 2].astype(jnp.bfloat16), jnp.uint16).astype(jnp.int32)
    hi = lax.bitcast_convert_type(
        table[:, EMB // 2:].astype(jnp.bfloat16), jnp.uint16).astype(jnp.int32)
    table_packed = ((hi << 16) | lo).reshape(-1)
    embedded = _emb_bag(text, table_packed)
    return _mlp(embedded, numeric_features,
                W1[:EMB], W1[EMB:], b1.reshape(1, -1),
                W2, b2.reshape(1, -1), W3, b3.reshape(1, -1))


# speedup vs baseline: 1.6084x; 1.6084x over previous
import functools

import jax
import jax.numpy as jnp
from jax import lax
from jax.experimental import pallas as pl
from jax.experimental.pallas import tpu as pltpu
from jax.experimental.pallas import tpu_sc as plsc

B = 16384
L = 50
VOCAB = 1000
EMB = 64

LP = 56
NUM_CORES = 2
NUM_SUBCORES = 16
NW = NUM_CORES * NUM_SUBCORES
BPW = B // NW
CH = 128
NCHUNK = BPW // CH


def _emb_bag(text_t, table_packed):
    mesh = plsc.VectorSubcoreMesh(core_axis_name="c", subcore_axis_name="s")

    @functools.partial(
        pl.kernel,
        mesh=mesh,
        out_type=jax.ShapeDtypeStruct((B, EMB), jnp.float32),
        scratch_types=[
            pltpu.VMEM((VOCAB * EMB // 2,), jnp.int32),
            pltpu.VMEM((2, CH, L), jnp.int32),
            pltpu.VMEM((2, CH, EMB), jnp.float32),
            pltpu.SemaphoreType.DMA,
            pltpu.SemaphoreType.DMA,
            pltpu.SemaphoreType.DMA,
            pltpu.SemaphoreType.DMA,
            pltpu.SemaphoreType.DMA,
        ],
    )
    def k(text_hbm, table_hbm, out_hbm, tab_v, txt_v, out_v,
          sem_tab, sin0, sin1, sout0, sout1):
        wid = lax.axis_index("s") * NUM_CORES + lax.axis_index("c")
        base = wid * BPW
        sins = (sin0, sin1)
        souts = (sout0, sout1)

        def in_copy(ci, buf):
            return pltpu.make_async_copy(
                text_hbm.at[pl.ds(base + ci * CH, CH)], txt_v.at[buf],
                sins[buf])

        def out_copy(ci, buf):
            return pltpu.make_async_copy(
                out_v.at[buf], out_hbm.at[pl.ds(base + ci * CH, CH)], souts[buf])

        tab_copy = pltpu.make_async_copy(table_hbm, tab_v, sem_tab)
        tab_copy.start()
        in_copy(0, 0).start()
        in_copy(1, 1).start()
        tab_copy.wait()

        def bag_body(buf):
            def body(b, _):
                toks = [txt_v[buf, b, pl.ds(c, 16)] for c in (0, 16, 32, 34)]
                acc = [jnp.zeros((16,), jnp.float32) for _ in range(8)]
                for j in range(L):
                    t = toks[j // 16][j % 16] if j < 48 else toks[3][j - 34]
                    off = t * (EMB // 2)
                    lo = tab_v[pl.ds(off, 16)]
                    hi = tab_v[pl.ds(off + 16, 16)]
                    p = (j & 1) * 4
                    acc[p] = acc[p] + lax.bitcast_convert_type(lo << 16, jnp.float32)
                    acc[p + 2] = acc[p + 2] + lax.bitcast_convert_type(lo, jnp.float32)
                    acc[p + 1] = acc[p + 1] + lax.bitcast_convert_type(hi << 16, jnp.float32)
                    acc[p + 3] = acc[p + 3] + lax.bitcast_convert_type(hi, jnp.float32)
                s = jnp.float32(1.0 / L)
                out_v[buf, b, pl.ds(0, 16)] = (acc[0] + acc[4]) * s
                out_v[buf, b, pl.ds(16, 16)] = (acc[1] + acc[5]) * s
                out_v[buf, b, pl.ds(32, 16)] = (acc[2] + acc[6]) * s
                out_v[buf, b, pl.ds(48, 16)] = (acc[3] + acc[7]) * s
                return 0
            return body

        def pair_body(i, _):
            for buf in range(2):
                ci = 2 * i + buf
                in_copy(ci, buf).wait()

                @pl.when(i > 0)
                def _():
                    out_copy(2 * (i - 1) + buf, buf).wait()

                lax.fori_loop(0, CH, bag_body(buf), 0)
                out_copy(ci, buf).start()

                @pl.when(i < NCHUNK // 2 - 1)
                def _():
                    in_copy(ci + 2, buf).start()
            return 0

        lax.fori_loop(0, NCHUNK // 2, pair_body, 0)
        out_copy(NCHUNK - 2, 0).wait()
        out_copy(NCHUNK - 1, 1).wait()

    return k(text_t, table_packed)


BLK = 4096


def _mlp_body(emb_ref, numt_ref, w1a_ref, w1b_ref, b1_ref, w2_ref, b2_ref,
              w3_ref, b3_ref, out_ref):
    bf = jnp.bfloat16
    f32 = jnp.float32
    h = lax.dot_general(w1a_ref[...].astype(bf), emb_ref[...].astype(bf),
                        (((0,), (1,)), ((), ())), preferred_element_type=f32)
    h = h + lax.dot_general(w1b_ref[...], numt_ref[...],
                            (((0,), (0,)), ((), ())), preferred_element_type=f32)
    h = jnp.maximum(h + b1_ref[...], 0.0)
    h = jnp.maximum(
        lax.dot_general(w2_ref[...].astype(bf), h.astype(bf),
                        (((0,), (0,)), ((), ())), preferred_element_type=f32)
        + b2_ref[...], 0.0)
    out_ref[...] = (
        lax.dot_general(w3_ref[...].astype(bf), h.astype(bf),
                        (((0,), (0,)), ((), ())), preferred_element_type=f32)
        + b3_ref[...])


def _mlp(embedded, numeric_t, W1a, W1b, b1, W2, b2, W3, b3):
    grid = (B // BLK,)
    full = lambda shape: pl.BlockSpec(shape, lambda i: (0, 0))
    out_t = pl.pallas_call(
        _mlp_body,
        grid=grid,
        in_specs=[
            pl.BlockSpec((BLK, EMB), lambda i: (i, 0)),
            pl.BlockSpec((2, BLK), lambda i: (0, i)),
            full((EMB, 128)),
            full((2, 128)),
            full((128, 1)),
            full((128, 64)),
            full((64, 1)),
            full((64, 32)),
            full((32, 1)),
        ],
        out_specs=pl.BlockSpec((32, BLK), lambda i: (0, i)),
        out_shape=jax.ShapeDtypeStruct((32, B), jnp.float32),
    )(embedded, numeric_t, W1a, W1b, b1, W2, b2, W3, b3)
    return out_t.T


def kernel(text, numeric_features, table, W1, b1, W2, b2, W3, b3):
    text = text.astype(jnp.int32)
    lo = lax.bitcast_convert_type(
        table[:, :EMB // 2].astype(jnp.bfloat16), jnp.uint16).astype(jnp.int32)
    hi = lax.bitcast_convert_type(
        table[:, EMB // 2:].astype(jnp.bfloat16), jnp.uint16).astype(jnp.int32)
    table_packed = ((hi << 16) | lo).reshape(-1)
    embedded = _emb_bag(text, table_packed)
    return _mlp(embedded, numeric_features.T,
                W1[:EMB], W1[EMB:], b1.reshape(-1, 1),
                W2, b2.reshape(-1, 1), W3, b3.reshape(-1, 1))


# speedup vs baseline: 1.6257x; 1.0108x over previous
import functools

import jax
import jax.numpy as jnp
from jax import lax
from jax.experimental import pallas as pl
from jax.experimental.pallas import tpu as pltpu
from jax.experimental.pallas import tpu_sc as plsc

B = 16384
L = 50
VOCAB = 1000
EMB = 64

LP = 56
NUM_CORES = 2
NUM_SUBCORES = 16
NW = NUM_CORES * NUM_SUBCORES
BPW = B // NW
CH = 128
NCHUNK = BPW // CH


def _emb_bag(text_t, table_packed):
    mesh = plsc.VectorSubcoreMesh(core_axis_name="c", subcore_axis_name="s")

    @functools.partial(
        pl.kernel,
        mesh=mesh,
        out_type=jax.ShapeDtypeStruct((B, EMB), jnp.float32),
        scratch_types=[
            pltpu.VMEM((VOCAB * EMB // 2,), jnp.int32),
            pltpu.VMEM((2, CH, L), jnp.int32),
            pltpu.VMEM((2, CH, EMB), jnp.float32),
            pltpu.SemaphoreType.DMA,
            pltpu.SemaphoreType.DMA,
            pltpu.SemaphoreType.DMA,
            pltpu.SemaphoreType.DMA,
            pltpu.SemaphoreType.DMA,
        ],
    )
    def k(text_hbm, table_hbm, out_hbm, tab_v, txt_v, out_v,
          sem_tab, sin0, sin1, sout0, sout1):
        wid = lax.axis_index("s") * NUM_CORES + lax.axis_index("c")
        base = wid * BPW
        sins = (sin0, sin1)
        souts = (sout0, sout1)

        def in_copy(ci, buf):
            return pltpu.make_async_copy(
                text_hbm.at[pl.ds(base + ci * CH, CH)], txt_v.at[buf],
                sins[buf])

        def out_copy(ci, buf):
            return pltpu.make_async_copy(
                out_v.at[buf], out_hbm.at[pl.ds(base + ci * CH, CH)], souts[buf])

        tab_copy = pltpu.make_async_copy(table_hbm, tab_v, sem_tab)
        tab_copy.start()
        in_copy(0, 0).start()
        in_copy(1, 1).start()
        tab_copy.wait()

        def bag_body(buf):
            def body(b, _):
                toks = [txt_v[buf, b, pl.ds(c, 16)] for c in (0, 16, 32, 34)]
                acc = [jnp.zeros((16,), jnp.float32) for _ in range(8)]
                for j in range(L):
                    t = toks[j // 16][j % 16] if j < 48 else toks[3][j - 34]
                    off = t * (EMB // 2)
                    lo = tab_v[pl.ds(off, 16)]
                    hi = tab_v[pl.ds(off + 16, 16)]
                    p = (j & 1) * 4
                    acc[p] = acc[p] + lax.bitcast_convert_type(lo << 16, jnp.float32)
                    acc[p + 2] = acc[p + 2] + lax.bitcast_convert_type(lo, jnp.float32)
                    acc[p + 1] = acc[p + 1] + lax.bitcast_convert_type(hi << 16, jnp.float32)
                    acc[p + 3] = acc[p + 3] + lax.bitcast_convert_type(hi, jnp.float32)
                s = jnp.float32(1.0 / L)
                out_v[buf, b, pl.ds(0, 16)] = (acc[0] + acc[4]) * s
                out_v[buf, b, pl.ds(16, 16)] = (acc[1] + acc[5]) * s
                out_v[buf, b, pl.ds(32, 16)] = (acc[2] + acc[6]) * s
                out_v[buf, b, pl.ds(48, 16)] = (acc[3] + acc[7]) * s
                return 0
            return body

        def pair_body(i, _):
            for buf in range(2):
                ci = 2 * i + buf
                in_copy(ci, buf).wait()

                @pl.when(i > 0)
                def _():
                    out_copy(2 * (i - 1) + buf, buf).wait()

                lax.fori_loop(0, CH, bag_body(buf), 0, unroll=2)
                out_copy(ci, buf).start()

                @pl.when(i < NCHUNK // 2 - 1)
                def _():
                    in_copy(ci + 2, buf).start()
            return 0

        lax.fori_loop(0, NCHUNK // 2, pair_body, 0)
        out_copy(NCHUNK - 2, 0).wait()
        out_copy(NCHUNK - 1, 1).wait()

    return k(text_t, table_packed)


BLK = 4096


def _mlp_body(emb_ref, numt_ref, w1a_ref, w1b_ref, b1_ref, w2_ref, b2_ref,
              w3_ref, b3_ref, out_ref):
    bf = jnp.bfloat16
    f32 = jnp.float32
    h = lax.dot_general(w1a_ref[...].astype(bf), emb_ref[...].astype(bf),
                        (((0,), (1,)), ((), ())), preferred_element_type=f32)
    h = h + lax.dot_general(w1b_ref[...], numt_ref[...],
                            (((0,), (0,)), ((), ())), preferred_element_type=f32)
    h = jnp.maximum(h + b1_ref[...], 0.0)
    h = jnp.maximum(
        lax.dot_general(w2_ref[...].astype(bf), h.astype(bf),
                        (((0,), (0,)), ((), ())), preferred_element_type=f32)
        + b2_ref[...], 0.0)
    out_ref[...] = (
        lax.dot_general(w3_ref[...].astype(bf), h.astype(bf),
                        (((0,), (0,)), ((), ())), preferred_element_type=f32)
        + b3_ref[...])


def _mlp(embedded, numeric_t, W1a, W1b, b1, W2, b2, W3, b3):
    grid = (B // BLK,)
    full = lambda shape: pl.BlockSpec(shape, lambda i: (0, 0))
    out_t = pl.pallas_call(
        _mlp_body,
        grid=grid,
        in_specs=[
            pl.BlockSpec((BLK, EMB), lambda i: (i, 0)),
            pl.BlockSpec((2, BLK), lambda i: (0, i)),
            full((EMB, 128)),
            full((2, 128)),
            full((128, 1)),
            full((128, 64)),
            full((64, 1)),
            full((64, 32)),
            full((32, 1)),
        ],
        out_specs=pl.BlockSpec((32, BLK), lambda i: (0, i)),
        out_shape=jax.ShapeDtypeStruct((32, B), jnp.float32),
    )(embedded, numeric_t, W1a, W1b, b1, W2, b2, W3, b3)
    return out_t.T


def kernel(text, numeric_features, table, W1, b1, W2, b2, W3, b3):
    text = text.astype(jnp.int32)
    lo = lax.bitcast_convert_type(
        table[:, :EMB // 2].astype(jnp.bfloat16), jnp.uint16).astype(jnp.int32)
    hi = lax.bitcast_convert_type(
        table[:, EMB // 2:].astype(jnp.bfloat16), jnp.uint16).astype(jnp.int32)
    table_packed = ((hi << 16) | lo).reshape(-1)
    embedded = _emb_bag(text, table_packed)
    return _mlp(embedded, numeric_features.T,
                W1[:EMB], W1[EMB:], b1.reshape(-1, 1),
                W2, b2.reshape(-1, 1), W3, b3.reshape(-1, 1))
